# baseline (speedup 1.0000x reference)
import functools

import jax
import jax.numpy as jnp
from jax import lax
from jax.experimental import pallas as pl
from jax.experimental.pallas import tpu as pltpu
from jax.experimental.pallas import tpu_sc as plsc

B, T, D = 4, 8192, 1024
TC_B = 3
BT = 1024

NC, NS, L = 2, 16, 16
NW = NC * NS
ROWS_W = T // NW
RB = 8
NB = ROWS_W // RB
RING = 4


def _tc_body(x_ref, pos_ref, o_ref):
    o_ref[...] = x_ref[...] + pos_ref[...]


def _tc_add(x, pos_table):
    return pl.pallas_call(
        _tc_body,
        grid=(T // BT, TC_B),
        in_specs=[
            pl.BlockSpec((1, BT, D), lambda i, j: (j, i, 0)),
            pl.BlockSpec((BT, D), lambda i, j: (i, 0)),
        ],
        out_specs=pl.BlockSpec((1, BT, D), lambda i, j: (j, i, 0)),
        out_shape=jax.ShapeDtypeStruct((TC_B, T, D), jnp.float32),
        compiler_params=pltpu.CompilerParams(
            dimension_semantics=("arbitrary", "arbitrary"),
        ),
    )(x, pos_table)


_mesh = plsc.VectorSubcoreMesh(core_axis_name="c", subcore_axis_name="s")

_scratch = (
    [pltpu.VMEM((RB, D), jnp.float32) for _ in range(2 * RING)]
    + [pltpu.SemaphoreType.DMA] * (3 * RING)
)


@functools.partial(
    pl.kernel,
    mesh=_mesh,
    out_type=jax.ShapeDtypeStruct((1, T, D), jnp.float32),
    scratch_types=_scratch,
)
def _sc_add(x_hbm, pos_hbm, out_hbm, *refs):
    xb = refs[0:RING]
    pb = refs[RING : 2 * RING]
    in_sem = refs[2 * RING : 3 * RING]
    pos_sem = refs[3 * RING : 4 * RING]
    out_sem = refs[4 * RING : 5 * RING]

    wid = lax.axis_index("s") * NC + lax.axis_index("c")
    base = wid * ROWS_W

    def start_in(i, j):
        r0 = base + i * RB
        pltpu.async_copy(x_hbm.at[TC_B, pl.ds(r0, RB)], xb[j], in_sem[j])
        pltpu.async_copy(pos_hbm.at[pl.ds(r0, RB)], pb[j], pos_sem[j])

    def wait_in(j):
        pltpu.make_async_copy(x_hbm.at[TC_B, pl.ds(base, RB)], xb[j], in_sem[j]).wait()
        pltpu.make_async_copy(pos_hbm.at[pl.ds(base, RB)], pb[j], pos_sem[j]).wait()

    def start_out(i, j):
        r0 = base + i * RB
        pltpu.async_copy(xb[j], out_hbm.at[0, pl.ds(r0, RB)], out_sem[j])

    def wait_out(j):
        pltpu.make_async_copy(xb[j], out_hbm.at[0, pl.ds(base, RB)], out_sem[j]).wait()

    start_in(0, 0)
    start_in(1, 1)

    def outer(ii, carry):
        i0 = ii * RING
        for j in range(RING):
            i = i0 + j
            jp = (j + 2) % RING

            @pl.when(i >= 2)
            def _():
                wait_out(jp)

            @pl.when(i + 2 < NB)
            def _():
                start_in(i + 2, jp)

            wait_in(j)

            def col(c8, cc):
                for u in range(8):
                    sl = pl.ds((c8 * 8 + u) * L, L)
                    for r in range(RB):
                        plsc.addupdate(xb[j].at[r, sl], pb[j][r, sl])
                return cc

            lax.fori_loop(0, D // L // 8, col, 0)
            start_out(i, j)
        return carry

    lax.fori_loop(0, NB // RING, outer, 0)

    wait_out((NB - 2) % RING)
    wait_out((NB - 1) % RING)


def kernel(x, pos_table):
    tc_out = _tc_add(x, pos_table)
    sc_out = _sc_add(x, pos_table)
    return jnp.concatenate([tc_out, sc_out], axis=0)


# speedup vs baseline: 1.0553x; 1.0553x over previous
import functools

import jax
import jax.numpy as jnp
from jax import lax
from jax.experimental import pallas as pl
from jax.experimental.pallas import tpu as pltpu
from jax.experimental.pallas import tpu_sc as plsc

B, T, D = 4, 8192, 1024
TC_T = 7168
BT = 1024

NC, NS, L = 2, 16, 16
NW = NC * NS
SC_T = T - TC_T
ROWS_W = SC_T // NW
RB = 8
NBB = ROWS_W // RB
NBLK = B * NBB
RING = 4


def _tc_body(x_ref, pos_ref, o_ref):
    o_ref[...] = x_ref[...] + pos_ref[...]


def _tc_add(x, pos_table):
    return pl.pallas_call(
        _tc_body,
        grid=(TC_T // BT, B),
        in_specs=[
            pl.BlockSpec((1, BT, D), lambda i, j: (j, i, 0)),
            pl.BlockSpec((BT, D), lambda i, j: (i, 0)),
        ],
        out_specs=pl.BlockSpec((1, BT, D), lambda i, j: (j, i, 0)),
        out_shape=jax.ShapeDtypeStruct((B, TC_T, D), jnp.float32),
        compiler_params=pltpu.CompilerParams(
            dimension_semantics=("arbitrary", "arbitrary"),
        ),
    )(x, pos_table)


_mesh = plsc.VectorSubcoreMesh(core_axis_name="c", subcore_axis_name="s")

_scratch = (
    [pltpu.VMEM((RB, D), jnp.float32) for _ in range(RING)]
    + [pltpu.VMEM((ROWS_W, D), jnp.float32)]
    + [pltpu.SemaphoreType.DMA] * (2 * RING + 1)
)


@functools.partial(
    pl.kernel,
    mesh=_mesh,
    out_type=jax.ShapeDtypeStruct((B, SC_T, D), jnp.float32),
    scratch_types=_scratch,
)
def _sc_add(x_hbm, pos_hbm, out_hbm, *refs):
    xb = refs[0:RING]
    pbuf = refs[RING]
    in_sem = refs[RING + 1 : 2 * RING + 1]
    out_sem = refs[2 * RING + 1 : 3 * RING + 1]
    pos_sem = refs[3 * RING + 1]

    wid = lax.axis_index("s") * NC + lax.axis_index("c")
    tail = wid * ROWS_W
    row0 = TC_T + tail

    def start_in(i, j):
        b, blk = divmod(i, NBB)
        pltpu.async_copy(
            x_hbm.at[b, pl.ds(row0 + blk * RB, RB)], xb[j], in_sem[j]
        )

    def wait_in(j):
        pltpu.make_async_copy(
            x_hbm.at[0, pl.ds(row0, RB)], xb[j], in_sem[j]
        ).wait()

    def start_out(i, j):
        b, blk = divmod(i, NBB)
        pltpu.async_copy(
            xb[j], out_hbm.at[b, pl.ds(tail + blk * RB, RB)], out_sem[j]
        )

    def wait_out(j):
        pltpu.make_async_copy(
            xb[j], out_hbm.at[0, pl.ds(tail, RB)], out_sem[j]
        ).wait()

    pltpu.async_copy(pos_hbm.at[pl.ds(row0, ROWS_W)], pbuf, pos_sem)

    start_in(0, 0)
    start_in(1, 1)

    pltpu.make_async_copy(pos_hbm.at[pl.ds(row0, ROWS_W)], pbuf, pos_sem).wait()

    for i in range(NBLK):
        j = i % RING
        jp = (i + 2) % RING

        if i >= 2:
            wait_out(jp)
        if i + 2 < NBLK:
            start_in(i + 2, jp)

        wait_in(j)

        blk = i % NBB

        def col(c8, cc, j=j, blk=blk):
            for u in range(8):
                sl = pl.ds((c8 * 8 + u) * L, L)
                for r in range(RB):
                    plsc.addupdate(xb[j].at[r, sl], pbuf[blk * RB + r, sl])
            return cc

        lax.fori_loop(0, D // L // 8, col, 0)
        start_out(i, j)

    wait_out((NBLK - 2) % RING)
    wait_out((NBLK - 1) % RING)


def kernel(x, pos_table):
    tc_out = _tc_add(x, pos_table)
    sc_out = _sc_add(x, pos_table)
    return jnp.concatenate([tc_out, sc_out], axis=1)


# speedup vs baseline: 2.2376x; 2.1203x over previous
import jax
import jax.numpy as jnp
from jax.experimental import pallas as pl
from jax.experimental.pallas import tpu as pltpu

BT = 2048


def _add_kernel(x_ref, pos_ref, o_ref):
    o_ref[...] = x_ref[...] + pos_ref[...]


def kernel(x, pos_table):
    b, t, d = x.shape
    grid = (t // BT, b)
    return pl.pallas_call(
        _add_kernel,
        grid=grid,
        in_specs=[
            pl.BlockSpec((1, BT, d), lambda i, j: (j, i, 0)),
            pl.BlockSpec((BT, d), lambda i, j: (i, 0)),
        ],
        out_specs=pl.BlockSpec((1, BT, d), lambda i, j: (j, i, 0)),
        out_shape=jax.ShapeDtypeStruct((b, t, d), x.dtype),
        compiler_params=pltpu.CompilerParams(
            dimension_semantics=("arbitrary", "arbitrary"),
        ),
    )(x, pos_table)
